# Initial kernel scaffold; baseline (speedup 1.0000x reference)
import jax
import jax.numpy as jnp
from jax.experimental import pallas as pl


def kernel(token_ids, weight):
    raise NotImplementedError("write your pallas kernel here")


# speedup vs baseline: 1.6853x; 1.6853x over previous
import functools

import jax
import jax.numpy as jnp
from jax import lax
from jax.experimental import pallas as pl
from jax.experimental.pallas import tpu as pltpu
from jax.experimental.pallas import tpu_sc as plsc

D = 64
CHUNK = 128


def _make_lookup(n_chunks_per_worker: int, total_chunks: int):
    info = plsc.get_sparse_core_info()
    nc, ns = info.num_cores, info.num_subcores

    mesh = plsc.VectorSubcoreMesh(core_axis_name="c", subcore_axis_name="s")

    @functools.partial(
        pl.kernel,
        mesh=mesh,
        compiler_params=pltpu.CompilerParams(use_tc_tiling_on_sc=False),
        out_type=jax.ShapeDtypeStruct((total_chunks * CHUNK, D), jnp.float32),
        scratch_types=[
            pltpu.VMEM((n_chunks_per_worker, CHUNK), jnp.int32),
            pltpu.VMEM((CHUNK, D), jnp.float32),
            pltpu.SemaphoreType.DMA,
        ],
    )
    def lookup(tok_hbm, table_hbm, out_hbm, idx_v, rows_v, gsem):
        wid = lax.axis_index("s") * nc + lax.axis_index("c")
        base_chunk = wid * n_chunks_per_worker
        pltpu.sync_copy(tok_hbm.at[pl.ds(base_chunk, n_chunks_per_worker)],
                        idx_v)

        def step(j, carry):
            pltpu.async_copy(table_hbm.at[idx_v.at[j]], rows_v, gsem).wait()
            pltpu.sync_copy(
                rows_v, out_hbm.at[pl.ds((base_chunk + j) * CHUNK, CHUNK)])
            return carry

        lax.fori_loop(0, n_chunks_per_worker, step, 0)

    return lookup


def kernel(token_ids, weight):
    b, s = token_ids.shape
    n = b * s
    assert n % CHUNK == 0
    total_chunks = n // CHUNK
    info = plsc.get_sparse_core_info()
    nw = info.num_cores * info.num_subcores
    assert total_chunks % nw == 0
    tok2d = token_ids.reshape(total_chunks, CHUNK)
    lookup = _make_lookup(total_chunks // nw, total_chunks)
    out = lookup(tok2d, weight)
    return out.reshape(b, s, D)


# speedup vs baseline: 1.8695x; 1.1092x over previous
import functools

import jax
import jax.numpy as jnp
from jax import lax
from jax.experimental import pallas as pl
from jax.experimental.pallas import tpu as pltpu
from jax.experimental.pallas import tpu_sc as plsc

D = 64
CHUNK = 128


def _make_lookup(n_chunks_per_worker: int, total_chunks: int):
    info = plsc.get_sparse_core_info()
    nc, ns = info.num_cores, info.num_subcores

    mesh = plsc.VectorSubcoreMesh(core_axis_name="c", subcore_axis_name="s")

    nb = 4
    assert n_chunks_per_worker % nb == 0
    nrounds = n_chunks_per_worker // nb

    @functools.partial(
        pl.kernel,
        mesh=mesh,
        compiler_params=pltpu.CompilerParams(use_tc_tiling_on_sc=False),
        out_type=jax.ShapeDtypeStruct((total_chunks * CHUNK, D), jnp.float32),
        scratch_types=[
            pltpu.VMEM((n_chunks_per_worker, CHUNK), jnp.int32),
            pltpu.VMEM((nb, CHUNK, D), jnp.float32),
            pltpu.SemaphoreType.DMA((nb,)),
            pltpu.SemaphoreType.DMA((nb,)),
        ],
    )
    def lookup(tok_hbm, table_hbm, out_hbm, idx_v, rows_v, gsem, ssem):
        wid = lax.axis_index("s") * nc + lax.axis_index("c")
        base_chunk = wid * n_chunks_per_worker
        pltpu.sync_copy(tok_hbm.at[pl.ds(base_chunk, n_chunks_per_worker)],
                        idx_v)

        def gather(j, b):
            return pltpu.async_copy(table_hbm.at[idx_v.at[j]], rows_v.at[b],
                                    gsem.at[b])

        def scatter(j, b):
            return pltpu.async_copy(
                rows_v.at[b],
                out_hbm.at[pl.ds((base_chunk + j) * CHUNK, CHUNK)],
                ssem.at[b])

        def wait_gather(j, b):
            pltpu.make_async_copy(table_hbm.at[idx_v.at[j]], rows_v.at[b],
                                  gsem.at[b]).wait()

        def wait_scatter(j, b):
            pltpu.make_async_copy(
                rows_v.at[b],
                out_hbm.at[pl.ds((base_chunk + j) * CHUNK, CHUNK)],
                ssem.at[b]).wait()

        for b in range(nb):
            gather(b, b)

        def round_fn(r, carry):
            j0 = r * nb
            for b in range(nb):
                wait_gather(j0 + b, b)
                scatter(j0 + b, b)
            for b in range(nb):
                wait_scatter(j0 + b, b)

                @pl.when(r + 1 < nrounds)
                def _():
                    gather(j0 + nb + b, b)
            return carry

        lax.fori_loop(0, nrounds, round_fn, 0)

    return lookup


def kernel(token_ids, weight):
    b, s = token_ids.shape
    n = b * s
    assert n % CHUNK == 0
    total_chunks = n // CHUNK
    info = plsc.get_sparse_core_info()
    nw = info.num_cores * info.num_subcores
    assert total_chunks % nw == 0
    tok2d = token_ids.reshape(total_chunks, CHUNK)
    lookup = _make_lookup(total_chunks // nw, total_chunks)
    out = lookup(tok2d, weight)
    return out.reshape(b, s, D)


# speedup vs baseline: 1.8718x; 1.0012x over previous
import functools

import jax
import jax.numpy as jnp
from jax import lax
from jax.experimental import pallas as pl
from jax.experimental.pallas import tpu as pltpu
from jax.experimental.pallas import tpu_sc as plsc

D = 64
CHUNK = 128


def _make_lookup(n_chunks_per_worker: int, total_chunks: int):
    info = plsc.get_sparse_core_info()
    nc, ns = info.num_cores, info.num_subcores

    mesh = plsc.VectorSubcoreMesh(core_axis_name="c", subcore_axis_name="s")

    nb = 8
    assert n_chunks_per_worker % nb == 0
    nrounds = n_chunks_per_worker // nb

    @functools.partial(
        pl.kernel,
        mesh=mesh,
        compiler_params=pltpu.CompilerParams(use_tc_tiling_on_sc=False),
        out_type=jax.ShapeDtypeStruct((total_chunks * CHUNK, D), jnp.float32),
        scratch_types=[
            pltpu.VMEM((n_chunks_per_worker, CHUNK), jnp.int32),
            pltpu.VMEM((nb, CHUNK, D), jnp.float32),
            pltpu.SemaphoreType.DMA((nb,)),
            pltpu.SemaphoreType.DMA((nb,)),
        ],
    )
    def lookup(tok_hbm, table_hbm, out_hbm, idx_v, rows_v, gsem, ssem):
        wid = lax.axis_index("s") * nc + lax.axis_index("c")
        base_chunk = wid * n_chunks_per_worker
        pltpu.sync_copy(tok_hbm.at[pl.ds(base_chunk, n_chunks_per_worker)],
                        idx_v)

        def gather(j, b):
            return pltpu.async_copy(table_hbm.at[idx_v.at[j]], rows_v.at[b],
                                    gsem.at[b])

        def scatter(j, b):
            return pltpu.async_copy(
                rows_v.at[b],
                out_hbm.at[pl.ds((base_chunk + j) * CHUNK, CHUNK)],
                ssem.at[b])

        def wait_gather(j, b):
            pltpu.make_async_copy(table_hbm.at[idx_v.at[j]], rows_v.at[b],
                                  gsem.at[b]).wait()

        def wait_scatter(j, b):
            pltpu.make_async_copy(
                rows_v.at[b],
                out_hbm.at[pl.ds((base_chunk + j) * CHUNK, CHUNK)],
                ssem.at[b]).wait()

        for b in range(nb):
            gather(b, b)

        def round_fn(r, carry):
            j0 = r * nb
            for b in range(nb):
                wait_gather(j0 + b, b)
                scatter(j0 + b, b)
            for b in range(nb):
                wait_scatter(j0 + b, b)

                @pl.when(r + 1 < nrounds)
                def _():
                    gather(j0 + nb + b, b)
            return carry

        lax.fori_loop(0, nrounds, round_fn, 0)

    return lookup


def kernel(token_ids, weight):
    b, s = token_ids.shape
    n = b * s
    assert n % CHUNK == 0
    total_chunks = n // CHUNK
    info = plsc.get_sparse_core_info()
    nw = info.num_cores * info.num_subcores
    assert total_chunks % nw == 0
    tok2d = token_ids.reshape(total_chunks, CHUNK)
    lookup = _make_lookup(total_chunks // nw, total_chunks)
    out = lookup(tok2d, weight)
    return out.reshape(b, s, D)
